# Initial kernel scaffold; baseline (speedup 1.0000x reference)
import jax
import jax.numpy as jnp
from jax.experimental import pallas as pl


def kernel(x, edge_index, W1, b1, Wmu, bmu, Wlv, blv, eps):
    raise NotImplementedError("write your pallas kernel here")


# speedup vs baseline: 8.7785x; 8.7785x over previous
import functools

import jax
import jax.numpy as jnp
from jax import lax
from jax.experimental import pallas as pl
from jax.experimental.pallas import tpu as pltpu
from jax.experimental.pallas import tpu_sc as plsc

_NC, _NS = 2, 16


def _chunk_size(n, cap=128):
    best = 0
    for c in range(8, cap + 1, 8):
        if n % c == 0:
            best = c
    assert best, n
    return best


def _sc_degree(dst, n_nodes):
    e = dst.shape[0]
    nw = _NC * _NS
    ew = e // nw
    assert ew * nw == e
    ch = _chunk_size(ew)
    nch = ew // ch
    rows = (n_nodes // _NS) // 8 * 8
    tail = n_nodes - _NS * rows
    z16 = jnp.zeros((n_nodes, 128), jnp.float32)
    ones16 = jnp.ones((ch, 128), jnp.float32)
    mesh = plsc.VectorSubcoreMesh(core_axis_name="c", subcore_axis_name="s")

    @functools.partial(
        pl.kernel,
        out_type=[jax.ShapeDtypeStruct((n_nodes, 128), jnp.float32)] * 2,
        mesh=mesh,
        scratch_types=[
            pltpu.VMEM((ch,), jnp.int32),
            pltpu.VMEM((ch, 128), jnp.float32),
            pltpu.VMEM_SHARED((n_nodes, 128), jnp.float32),
        ],
    )
    def deg_kernel(dst_hbm, z_hbm, ones_hbm, out0, out1, idx_v, ones_v, acc):
        c = lax.axis_index("c")
        s = lax.axis_index("s")
        wid = s * _NC + c

        pltpu.sync_copy(ones_hbm, ones_v)
        pltpu.sync_copy(z_hbm.at[pl.ds(s * rows, rows)], acc.at[pl.ds(s * rows, rows)])
        if tail:
            @pl.when(s == _NS - 1)
            def _():
                pltpu.sync_copy(z_hbm.at[pl.ds(_NS * rows, tail)],
                                acc.at[pl.ds(_NS * rows, tail)])
        plsc.subcore_barrier()

        base0 = wid * ew

        def body(k, carry):
            pltpu.sync_copy(dst_hbm.at[pl.ds(base0 + k * ch, ch)], idx_v)
            pltpu.sync_copy(ones_v, acc.at[idx_v], add=True)
            return carry

        lax.fori_loop(0, nch, body, 0)
        plsc.subcore_barrier()

        def writeback(out):
            pltpu.sync_copy(acc.at[pl.ds(s * rows, rows)],
                            out.at[pl.ds(s * rows, rows)])
            if tail:
                @pl.when(s == _NS - 1)
                def _():
                    pltpu.sync_copy(acc.at[pl.ds(_NS * rows, tail)],
                                    out.at[pl.ds(_NS * rows, tail)])

        @pl.when(c == 0)
        def _():
            writeback(out0)

        @pl.when(c == 1)
        def _():
            writeback(out1)

    return deg_kernel(dst, z16, ones16)


def _sc_aggregate(tab_a, tab_b, src, dst):
    n_nodes, dh = tab_a.shape
    e = src.shape[0]
    ew = e // _NS
    assert ew * _NS == e
    ch = _chunk_size(ew)
    nch = ew // ch
    rows = (n_nodes // _NS) // 8 * 8
    tail = n_nodes - _NS * rows
    z = jnp.zeros((n_nodes, dh), jnp.float32)
    mesh = plsc.VectorSubcoreMesh(core_axis_name="c", subcore_axis_name="s")

    @functools.partial(
        pl.kernel,
        out_type=[jax.ShapeDtypeStruct((n_nodes, dh), jnp.float32)] * 2,
        mesh=mesh,
        scratch_types=[
            pltpu.VMEM((ch,), jnp.int32),
            pltpu.VMEM((ch,), jnp.int32),
            pltpu.VMEM((ch, dh), jnp.float32),
            pltpu.VMEM_SHARED((n_nodes, dh), jnp.float32),
            pltpu.SemaphoreType.DMA,
        ],
    )
    def agg_kernel(ta, tb, src_hbm, dst_hbm, z_hbm, out_a, out_b,
                   sidx, didx, rows_v, acc, sem):
        c = lax.axis_index("c")
        s = lax.axis_index("s")

        def run(tab, out):
            pltpu.sync_copy(z_hbm.at[pl.ds(s * rows, rows)],
                            acc.at[pl.ds(s * rows, rows)])
            if tail:
                @pl.when(s == _NS - 1)
                def _():
                    pltpu.sync_copy(z_hbm.at[pl.ds(_NS * rows, tail)],
                                    acc.at[pl.ds(_NS * rows, tail)])
            plsc.subcore_barrier()
            base0 = s * ew

            def body(k, carry):
                b = base0 + k * ch
                pltpu.sync_copy(src_hbm.at[pl.ds(b, ch)], sidx)
                pltpu.async_copy(tab.at[sidx], rows_v, sem).wait()
                pltpu.sync_copy(dst_hbm.at[pl.ds(b, ch)], didx)
                pltpu.sync_copy(rows_v, acc.at[didx], add=True)
                return carry

            lax.fori_loop(0, nch, body, 0)
            plsc.subcore_barrier()
            pltpu.sync_copy(acc.at[pl.ds(s * rows, rows)],
                            out.at[pl.ds(s * rows, rows)])
            if tail:
                @pl.when(s == _NS - 1)
                def _():
                    pltpu.sync_copy(acc.at[pl.ds(_NS * rows, tail)],
                                    out.at[pl.ds(_NS * rows, tail)])

        @pl.when(c == 0)
        def _():
            run(ta, out_a)

        @pl.when(c == 1)
        def _():
            run(tb, out_b)

    return agg_kernel(tab_a, tab_b, src, dst, z)


def _sc_aggregate_edges(tab, src, dst):
    n_nodes, dh = tab.shape
    assert dh % 128 == 0
    e = src.shape[0]
    eh = e // _NC
    ew = eh // _NS
    assert ew * _NS * _NC == e
    ch = _chunk_size(ew)
    nch = ew // ch
    rows = (n_nodes // _NS) // 8 * 8
    tail = n_nodes - _NS * rows
    z = jnp.zeros((n_nodes, dh), jnp.float32)
    mesh = plsc.VectorSubcoreMesh(core_axis_name="c", subcore_axis_name="s")

    @functools.partial(
        pl.kernel,
        out_type=[jax.ShapeDtypeStruct((n_nodes, dh), jnp.float32)] * 2,
        mesh=mesh,
        scratch_types=[
            pltpu.VMEM((ch,), jnp.int32),
            pltpu.VMEM((ch,), jnp.int32),
            pltpu.VMEM((ch, dh), jnp.float32),
            pltpu.VMEM_SHARED((n_nodes, dh), jnp.float32),
            pltpu.SemaphoreType.DMA,
        ],
    )
    def agg_kernel(tab_hbm, src_hbm, dst_hbm, z_hbm, out0, out1,
                   sidx, didx, rows_v, acc, sem):
        c = lax.axis_index("c")
        s = lax.axis_index("s")
        pltpu.sync_copy(z_hbm.at[pl.ds(s * rows, rows)],
                        acc.at[pl.ds(s * rows, rows)])
        if tail:
            @pl.when(s == _NS - 1)
            def _():
                pltpu.sync_copy(z_hbm.at[pl.ds(_NS * rows, tail)],
                                acc.at[pl.ds(_NS * rows, tail)])
        plsc.subcore_barrier()
        base0 = c * eh + s * ew

        def body(k, carry):
            b = base0 + k * ch
            pltpu.sync_copy(src_hbm.at[pl.ds(b, ch)], sidx)
            pltpu.async_copy(tab_hbm.at[sidx], rows_v, sem).wait()
            pltpu.sync_copy(dst_hbm.at[pl.ds(b, ch)], didx)
            pltpu.sync_copy(rows_v, acc.at[didx], add=True)
            return carry

        lax.fori_loop(0, nch, body, 0)
        plsc.subcore_barrier()

        def writeback(out):
            pltpu.sync_copy(acc.at[pl.ds(s * rows, rows)],
                            out.at[pl.ds(s * rows, rows)])
            if tail:
                @pl.when(s == _NS - 1)
                def _():
                    pltpu.sync_copy(acc.at[pl.ds(_NS * rows, tail)],
                                    out.at[pl.ds(_NS * rows, tail)])

        @pl.when(c == 0)
        def _():
            writeback(out0)

        @pl.when(c == 1)
        def _():
            writeback(out1)

    return agg_kernel(tab, src, dst, z)


def _dinv_from(d0, d1):
    deg = d0[:, :1] + d1[:, :1] + 1.0
    return lax.rsqrt(deg)


def _tc_pre(x, w1, d0, d1):
    n, f = x.shape
    hid = w1.shape[1]
    bn = 1000
    grid = n // bn

    def body(x_r, w_r, d0_r, d1_r, oa_r, ob_r):
        dinv = _dinv_from(d0_r[...], d1_r[...])
        r = jnp.dot(x_r[...], w_r[...], preferred_element_type=jnp.float32)
        hs = dinv * r
        oa_r[...] = hs[:, : hid // 2]
        ob_r[...] = hs[:, hid // 2:]

    return pl.pallas_call(
        body,
        grid=(grid,),
        in_specs=[
            pl.BlockSpec((bn, f), lambda i: (i, 0)),
            pl.BlockSpec((f, hid), lambda i: (0, 0)),
            pl.BlockSpec((bn, 128), lambda i: (i, 0)),
            pl.BlockSpec((bn, 128), lambda i: (i, 0)),
        ],
        out_specs=[
            pl.BlockSpec((bn, hid // 2), lambda i: (i, 0)),
            pl.BlockSpec((bn, hid // 2), lambda i: (i, 0)),
        ],
        out_shape=[jax.ShapeDtypeStruct((n, hid // 2), jnp.float32)] * 2,
    )(x, w1, d0, d1)


def _tc_mid(a1a, a1b, hs0a, hs0b, d0, d1, b1, wc):
    n = a1a.shape[0]
    hid = 2 * a1a.shape[1]
    dl2 = wc.shape[1]
    bn = 1000
    grid = n // bn

    def body(aa_r, ab_r, ha_r, hb_r, d0_r, d1_r, b1_r, w_r, o_r):
        dinv = _dinv_from(d0_r[...], d1_r[...])
        agg = jnp.concatenate([aa_r[...], ab_r[...]], axis=1)
        hs0 = jnp.concatenate([ha_r[...], hb_r[...]], axis=1)
        h = jnp.maximum(dinv * (agg + hs0) + b1_r[...], 0.0)
        hc = jnp.dot(h, w_r[...], preferred_element_type=jnp.float32)
        o_r[...] = dinv * hc

    return pl.pallas_call(
        body,
        grid=(grid,),
        in_specs=[
            pl.BlockSpec((bn, hid // 2), lambda i: (i, 0)),
            pl.BlockSpec((bn, hid // 2), lambda i: (i, 0)),
            pl.BlockSpec((bn, hid // 2), lambda i: (i, 0)),
            pl.BlockSpec((bn, hid // 2), lambda i: (i, 0)),
            pl.BlockSpec((bn, 128), lambda i: (i, 0)),
            pl.BlockSpec((bn, 128), lambda i: (i, 0)),
            pl.BlockSpec((1, hid), lambda i: (0, 0)),
            pl.BlockSpec((hid, dl2), lambda i: (0, 0)),
        ],
        out_specs=pl.BlockSpec((bn, dl2), lambda i: (i, 0)),
        out_shape=jax.ShapeDtypeStruct((n, dl2), jnp.float32),
    )(a1a, a1b, hs0a, hs0b, d0, d1, b1, wc)


def _tc_post(p0, p1, hsc, d0, d1, bmu, blv, eps):
    n, dl2 = p0.shape
    dl = dl2 // 2
    bn = 1000
    grid = n // bn

    def body(p0_r, p1_r, h_r, d0_r, d1_r, bmu_r, blv_r, eps_r, o_r):
        dinv = _dinv_from(d0_r[...], d1_r[...])
        g = dinv * (p0_r[...] + p1_r[...] + h_r[...])
        mu = g[:, :dl] + bmu_r[...]
        logvar = g[:, dl:] + blv_r[...]
        z = mu + eps_r[...] * jnp.exp(0.5 * logvar)
        o_r[...] = jnp.concatenate([z, mu, logvar], axis=1)

    return pl.pallas_call(
        body,
        grid=(grid,),
        in_specs=[
            pl.BlockSpec((bn, dl2), lambda i: (i, 0)),
            pl.BlockSpec((bn, dl2), lambda i: (i, 0)),
            pl.BlockSpec((bn, dl2), lambda i: (i, 0)),
            pl.BlockSpec((bn, 128), lambda i: (i, 0)),
            pl.BlockSpec((bn, 128), lambda i: (i, 0)),
            pl.BlockSpec((1, dl), lambda i: (0, 0)),
            pl.BlockSpec((1, dl), lambda i: (0, 0)),
            pl.BlockSpec((bn, dl), lambda i: (i, 0)),
        ],
        out_specs=pl.BlockSpec((bn, 3 * dl), lambda i: (i, 0)),
        out_shape=jax.ShapeDtypeStruct((n, 3 * dl), jnp.float32),
    )(p0, p1, hsc, d0, d1, bmu, blv, eps)


def kernel(x, edge_index, W1, b1, Wmu, bmu, Wlv, blv, eps):
    src = edge_index[0]
    dst = edge_index[1]
    n = x.shape[0]
    d0, d1 = _sc_degree(dst, n)
    hs0a, hs0b = _tc_pre(x, W1, d0, d1)
    a1a, a1b = _sc_aggregate(hs0a, hs0b, src, dst)
    wc = jnp.concatenate([Wmu, Wlv], axis=1)
    hsc = _tc_mid(a1a, a1b, hs0a, hs0b, d0, d1, b1.reshape(1, -1), wc)
    p0, p1 = _sc_aggregate_edges(hsc, src, dst)
    return _tc_post(p0, p1, hsc, d0, d1,
                    bmu.reshape(1, -1), blv.reshape(1, -1), eps)


# speedup vs baseline: 10.5068x; 1.1969x over previous
import functools

import jax
import jax.numpy as jnp
from jax import lax
from jax.experimental import pallas as pl
from jax.experimental.pallas import tpu as pltpu
from jax.experimental.pallas import tpu_sc as plsc

_NC, _NS = 2, 16


def _chunk_size(n, cap=128):
    best = 0
    for c in range(8, cap + 1, 8):
        if n % c == 0:
            best = c
    assert best, n
    return best


def _sc_degree(dst, n_nodes):
    e = dst.shape[0]
    nw = _NC * _NS
    ew = e // nw
    assert ew * nw == e
    ch = _chunk_size(ew)
    nch = ew // ch
    rows = (n_nodes // _NS) // 8 * 8
    tail = n_nodes - _NS * rows
    z16 = jnp.zeros((n_nodes, 128), jnp.float32)
    ones16 = jnp.ones((ch, 128), jnp.float32)
    mesh = plsc.VectorSubcoreMesh(core_axis_name="c", subcore_axis_name="s")

    @functools.partial(
        pl.kernel,
        out_type=[jax.ShapeDtypeStruct((n_nodes, 128), jnp.float32)] * 2,
        mesh=mesh,
        scratch_types=[
            pltpu.VMEM((ch,), jnp.int32),
            pltpu.VMEM((ch, 128), jnp.float32),
            pltpu.VMEM_SHARED((n_nodes, 128), jnp.float32),
        ],
    )
    def deg_kernel(dst_hbm, z_hbm, ones_hbm, out0, out1, idx_v, ones_v, acc):
        c = lax.axis_index("c")
        s = lax.axis_index("s")
        wid = s * _NC + c

        pltpu.sync_copy(ones_hbm, ones_v)
        pltpu.sync_copy(z_hbm.at[pl.ds(s * rows, rows)], acc.at[pl.ds(s * rows, rows)])
        if tail:
            @pl.when(s == _NS - 1)
            def _():
                pltpu.sync_copy(z_hbm.at[pl.ds(_NS * rows, tail)],
                                acc.at[pl.ds(_NS * rows, tail)])
        plsc.subcore_barrier()

        base0 = wid * ew

        def body(k, carry):
            pltpu.sync_copy(dst_hbm.at[pl.ds(base0 + k * ch, ch)], idx_v)
            pltpu.sync_copy(ones_v, acc.at[idx_v], add=True)
            return carry

        lax.fori_loop(0, nch, body, 0)
        plsc.subcore_barrier()

        def writeback(out):
            pltpu.sync_copy(acc.at[pl.ds(s * rows, rows)],
                            out.at[pl.ds(s * rows, rows)])
            if tail:
                @pl.when(s == _NS - 1)
                def _():
                    pltpu.sync_copy(acc.at[pl.ds(_NS * rows, tail)],
                                    out.at[pl.ds(_NS * rows, tail)])

        @pl.when(c == 0)
        def _():
            writeback(out0)

        @pl.when(c == 1)
        def _():
            writeback(out1)

    return deg_kernel(dst, z16, ones16)


def _agg_pipeline(tab, src_hbm, dst_hbm, acc, sidx, didx, rows_v, gsem,
                  ebase, ew, ch, nb):
    nch = ew // ch
    ngroups = nch // nb

    def issue(slot, k):
        b = ebase + k * ch
        pltpu.sync_copy(src_hbm.at[pl.ds(b, ch)], sidx[slot])
        pltpu.sync_copy(dst_hbm.at[pl.ds(b, ch)], didx[slot])
        pltpu.async_copy(tab.at[sidx[slot]], rows_v[slot], gsem[slot])

    def drain_scatter(slot):
        pltpu.make_async_copy(tab.at[sidx[slot]], rows_v[slot],
                              gsem[slot]).wait()
        pltpu.sync_copy(rows_v[slot], acc.at[didx[slot]], add=True)

    for b in range(nb):
        issue(b, b)

    def group(g, carry):
        for b in range(nb):
            drain_scatter(b)
            issue(b, g * nb + b + nb)
        return carry

    lax.fori_loop(0, ngroups - 1, group, 0)
    for b in range(nb):
        drain_scatter(b)


def _sc_aggregate(tab_a, tab_b, src, dst):
    n_nodes, dh = tab_a.shape
    e = src.shape[0]
    ew = e // _NS
    assert ew * _NS == e
    nb = 5
    ch = _chunk_size(ew, 40)
    assert (ew // ch) % nb == 0
    rows = (n_nodes // _NS) // 8 * 8
    tail = n_nodes - _NS * rows
    z = jnp.zeros((n_nodes, dh), jnp.float32)
    mesh = plsc.VectorSubcoreMesh(core_axis_name="c", subcore_axis_name="s")

    @functools.partial(
        pl.kernel,
        out_type=[jax.ShapeDtypeStruct((n_nodes, dh), jnp.float32)] * 2,
        mesh=mesh,
        scratch_types=[
            [pltpu.VMEM((ch,), jnp.int32)] * nb,
            [pltpu.VMEM((ch,), jnp.int32)] * nb,
            [pltpu.VMEM((ch, dh), jnp.float32)] * nb,
            [pltpu.SemaphoreType.DMA] * nb,
            pltpu.VMEM_SHARED((n_nodes, dh), jnp.float32),
        ],
    )
    def agg_kernel(ta, tb, src_hbm, dst_hbm, z_hbm, out_a, out_b,
                   sidx, didx, rows_v, gsem, acc):
        c = lax.axis_index("c")
        s = lax.axis_index("s")

        def run(tab, out):
            pltpu.sync_copy(z_hbm.at[pl.ds(s * rows, rows)],
                            acc.at[pl.ds(s * rows, rows)])
            if tail:
                @pl.when(s == _NS - 1)
                def _():
                    pltpu.sync_copy(z_hbm.at[pl.ds(_NS * rows, tail)],
                                    acc.at[pl.ds(_NS * rows, tail)])
            plsc.subcore_barrier()
            _agg_pipeline(tab, src_hbm, dst_hbm, acc, sidx, didx, rows_v,
                          gsem, s * ew, ew, ch, nb)
            plsc.subcore_barrier()
            pltpu.sync_copy(acc.at[pl.ds(s * rows, rows)],
                            out.at[pl.ds(s * rows, rows)])
            if tail:
                @pl.when(s == _NS - 1)
                def _():
                    pltpu.sync_copy(acc.at[pl.ds(_NS * rows, tail)],
                                    out.at[pl.ds(_NS * rows, tail)])

        @pl.when(c == 0)
        def _():
            run(ta, out_a)

        @pl.when(c == 1)
        def _():
            run(tb, out_b)

    return agg_kernel(tab_a, tab_b, src, dst, z)


def _sc_aggregate_edges(tab, src, dst):
    n_nodes, dh = tab.shape
    assert dh % 128 == 0
    e = src.shape[0]
    eh = e // _NC
    ew = eh // _NS
    assert ew * _NS * _NC == e
    nb = 5
    ch = _chunk_size(ew, 40)
    assert (ew // ch) % nb == 0
    rows = (n_nodes // _NS) // 8 * 8
    tail = n_nodes - _NS * rows
    z = jnp.zeros((n_nodes, dh), jnp.float32)
    mesh = plsc.VectorSubcoreMesh(core_axis_name="c", subcore_axis_name="s")

    @functools.partial(
        pl.kernel,
        out_type=[jax.ShapeDtypeStruct((n_nodes, dh), jnp.float32)] * 2,
        mesh=mesh,
        scratch_types=[
            [pltpu.VMEM((ch,), jnp.int32)] * nb,
            [pltpu.VMEM((ch,), jnp.int32)] * nb,
            [pltpu.VMEM((ch, dh), jnp.float32)] * nb,
            [pltpu.SemaphoreType.DMA] * nb,
            pltpu.VMEM_SHARED((n_nodes, dh), jnp.float32),
        ],
    )
    def agg_kernel(tab_hbm, src_hbm, dst_hbm, z_hbm, out0, out1,
                   sidx, didx, rows_v, gsem, acc):
        c = lax.axis_index("c")
        s = lax.axis_index("s")
        pltpu.sync_copy(z_hbm.at[pl.ds(s * rows, rows)],
                        acc.at[pl.ds(s * rows, rows)])
        if tail:
            @pl.when(s == _NS - 1)
            def _():
                pltpu.sync_copy(z_hbm.at[pl.ds(_NS * rows, tail)],
                                acc.at[pl.ds(_NS * rows, tail)])
        plsc.subcore_barrier()
        _agg_pipeline(tab_hbm, src_hbm, dst_hbm, acc, sidx, didx, rows_v,
                      gsem, c * eh + s * ew, ew, ch, nb)
        plsc.subcore_barrier()

        def writeback(out):
            pltpu.sync_copy(acc.at[pl.ds(s * rows, rows)],
                            out.at[pl.ds(s * rows, rows)])
            if tail:
                @pl.when(s == _NS - 1)
                def _():
                    pltpu.sync_copy(acc.at[pl.ds(_NS * rows, tail)],
                                    out.at[pl.ds(_NS * rows, tail)])

        @pl.when(c == 0)
        def _():
            writeback(out0)

        @pl.when(c == 1)
        def _():
            writeback(out1)

    return agg_kernel(tab, src, dst, z)


def _dinv_from(d0, d1):
    deg = d0[:, :1] + d1[:, :1] + 1.0
    return lax.rsqrt(deg)


def _tc_pre(x, w1, d0, d1):
    n, f = x.shape
    hid = w1.shape[1]
    bn = 1000
    grid = n // bn

    def body(x_r, w_r, d0_r, d1_r, oa_r, ob_r):
        dinv = _dinv_from(d0_r[...], d1_r[...])
        r = jnp.dot(x_r[...], w_r[...], preferred_element_type=jnp.float32)
        hs = dinv * r
        oa_r[...] = hs[:, : hid // 2]
        ob_r[...] = hs[:, hid // 2:]

    return pl.pallas_call(
        body,
        grid=(grid,),
        in_specs=[
            pl.BlockSpec((bn, f), lambda i: (i, 0)),
            pl.BlockSpec((f, hid), lambda i: (0, 0)),
            pl.BlockSpec((bn, 128), lambda i: (i, 0)),
            pl.BlockSpec((bn, 128), lambda i: (i, 0)),
        ],
        out_specs=[
            pl.BlockSpec((bn, hid // 2), lambda i: (i, 0)),
            pl.BlockSpec((bn, hid // 2), lambda i: (i, 0)),
        ],
        out_shape=[jax.ShapeDtypeStruct((n, hid // 2), jnp.float32)] * 2,
    )(x, w1, d0, d1)


def _tc_mid(a1a, a1b, hs0a, hs0b, d0, d1, b1, wc):
    n = a1a.shape[0]
    hid = 2 * a1a.shape[1]
    dl2 = wc.shape[1]
    bn = 1000
    grid = n // bn

    def body(aa_r, ab_r, ha_r, hb_r, d0_r, d1_r, b1_r, w_r, o_r):
        dinv = _dinv_from(d0_r[...], d1_r[...])
        agg = jnp.concatenate([aa_r[...], ab_r[...]], axis=1)
        hs0 = jnp.concatenate([ha_r[...], hb_r[...]], axis=1)
        h = jnp.maximum(dinv * (agg + hs0) + b1_r[...], 0.0)
        hc = jnp.dot(h, w_r[...], preferred_element_type=jnp.float32)
        o_r[...] = dinv * hc

    return pl.pallas_call(
        body,
        grid=(grid,),
        in_specs=[
            pl.BlockSpec((bn, hid // 2), lambda i: (i, 0)),
            pl.BlockSpec((bn, hid // 2), lambda i: (i, 0)),
            pl.BlockSpec((bn, hid // 2), lambda i: (i, 0)),
            pl.BlockSpec((bn, hid // 2), lambda i: (i, 0)),
            pl.BlockSpec((bn, 128), lambda i: (i, 0)),
            pl.BlockSpec((bn, 128), lambda i: (i, 0)),
            pl.BlockSpec((1, hid), lambda i: (0, 0)),
            pl.BlockSpec((hid, dl2), lambda i: (0, 0)),
        ],
        out_specs=pl.BlockSpec((bn, dl2), lambda i: (i, 0)),
        out_shape=jax.ShapeDtypeStruct((n, dl2), jnp.float32),
    )(a1a, a1b, hs0a, hs0b, d0, d1, b1, wc)


def _tc_post(p0, p1, hsc, d0, d1, bmu, blv, eps):
    n, dl2 = p0.shape
    dl = dl2 // 2
    bn = 1000
    grid = n // bn

    def body(p0_r, p1_r, h_r, d0_r, d1_r, bmu_r, blv_r, eps_r, o_r):
        dinv = _dinv_from(d0_r[...], d1_r[...])
        g = dinv * (p0_r[...] + p1_r[...] + h_r[...])
        mu = g[:, :dl] + bmu_r[...]
        logvar = g[:, dl:] + blv_r[...]
        z = mu + eps_r[...] * jnp.exp(0.5 * logvar)
        o_r[...] = jnp.concatenate([z, mu, logvar], axis=1)

    return pl.pallas_call(
        body,
        grid=(grid,),
        in_specs=[
            pl.BlockSpec((bn, dl2), lambda i: (i, 0)),
            pl.BlockSpec((bn, dl2), lambda i: (i, 0)),
            pl.BlockSpec((bn, dl2), lambda i: (i, 0)),
            pl.BlockSpec((bn, 128), lambda i: (i, 0)),
            pl.BlockSpec((bn, 128), lambda i: (i, 0)),
            pl.BlockSpec((1, dl), lambda i: (0, 0)),
            pl.BlockSpec((1, dl), lambda i: (0, 0)),
            pl.BlockSpec((bn, dl), lambda i: (i, 0)),
        ],
        out_specs=pl.BlockSpec((bn, 3 * dl), lambda i: (i, 0)),
        out_shape=jax.ShapeDtypeStruct((n, 3 * dl), jnp.float32),
    )(p0, p1, hsc, d0, d1, bmu, blv, eps)


def kernel(x, edge_index, W1, b1, Wmu, bmu, Wlv, blv, eps):
    src = edge_index[0]
    dst = edge_index[1]
    n = x.shape[0]
    d0, d1 = _sc_degree(dst, n)
    hs0a, hs0b = _tc_pre(x, W1, d0, d1)
    a1a, a1b = _sc_aggregate(hs0a, hs0b, src, dst)
    wc = jnp.concatenate([Wmu, Wlv], axis=1)
    hsc = _tc_mid(a1a, a1b, hs0a, hs0b, d0, d1, b1.reshape(1, -1), wc)
    p0, p1 = _sc_aggregate_edges(hsc, src, dst)
    return _tc_post(p0, p1, hsc, d0, d1,
                    bmu.reshape(1, -1), blv.reshape(1, -1), eps)
